# baseline (speedup 1.0000x reference)
import functools

import jax
import jax.numpy as jnp
from jax import lax
from jax.experimental import pallas as pl
from jax.experimental.pallas import tpu as pltpu
from jax.experimental.pallas import tpu_sc as plsc

N = 10000
E = 160000
H = 512
D_IN = 255
NC, NS = 2, 16
NW = NC * NS
GCH = 128
SCH = 80
SPT = 125
CG = 128
NG = H // CG
NPT = 624
BN = 1000
BE = 800


def _ln(h, g, b):
    mu = jnp.mean(h, axis=-1, keepdims=True)
    var = jnp.mean((h - mu) ** 2, axis=-1, keepdims=True)
    return (h - mu) * lax.rsqrt(var + 1e-5) * g + b


def _node_body(x_ref, w1a_ref, embed_ref, w1bT_ref, b1_ref, p_ref):
    xb = x_ref[...]
    q = jnp.dot(embed_ref[...], w1bT_ref[...],
                preferred_element_type=jnp.float32)
    et = xb[:, 255].astype(jnp.int32).reshape(BN, 1)
    onehot = (et == lax.broadcasted_iota(jnp.int32, (1, 8), 1)).astype(jnp.float32)
    p = jnp.dot(xb, w1a_ref[...], preferred_element_type=jnp.float32)
    p = p + jnp.dot(onehot, q, preferred_element_type=jnp.float32)
    p_ref[...] = (p + b1_ref[0:1, :]).astype(jnp.bfloat16)


def _stats(r):
    mu = jnp.mean(r, axis=-1, keepdims=True)
    m2 = jnp.mean(r * r, axis=-1, keepdims=True)
    inv = lax.rsqrt(m2 - mu * mu + 1e-5)
    return mu, inv


def _mlp_body(g_ref, attr_ref, w1cT_ref, w2T_ref, w3T_ref, vecs_ref, h3_ref):
    gw = g_ref[...]
    glo = pltpu.unpack_elementwise(gw, index=0, packed_dtype=jnp.bfloat16,
                                   unpacked_dtype=jnp.float32)
    ghi = pltpu.unpack_elementwise(gw, index=1, packed_dtype=jnp.bfloat16,
                                   unpacked_dtype=jnp.float32)
    gfull = jnp.concatenate([glo, ghi], axis=-1)
    pre1 = gfull + jnp.dot(attr_ref[...], w1cT_ref[...],
                           preferred_element_type=jnp.float32)
    r = jax.nn.relu(pre1)
    mu, inv = _stats(r)
    t = jnp.dot(r.astype(jnp.bfloat16), w2T_ref[...],
                preferred_element_type=jnp.float32)
    pre2 = t * inv - (mu * inv) * vecs_ref[1:2, :] + vecs_ref[0:1, :]
    r = jax.nn.relu(pre2)
    mu, inv = _stats(r)
    t = jnp.dot(r.astype(jnp.bfloat16), w3T_ref[...],
                preferred_element_type=jnp.float32)
    pre3 = t * inv - (mu * inv) * vecs_ref[3:4, :] + vecs_ref[2:3, :]
    r = jax.nn.relu(pre3)
    mu, inv = _stats(r)
    h3_ref[...] = (r - mu) * (inv * vecs_ref[4:5, :]) + vecs_ref[5:6, :]


_sc_mesh = plsc.VectorSubcoreMesh(core_axis_name="c", subcore_axis_name="s")


def _make_gather(n_edges, per_w, rem, steps):
    @functools.partial(
        pl.kernel,
        out_type=jax.ShapeDtypeStruct((n_edges, H // 2), jnp.int32),
        mesh=_sc_mesh,
        scratch_types=[
            pltpu.VMEM((GCH,), jnp.int32),
            pltpu.VMEM((GCH,), jnp.int32),
            pltpu.VMEM((GCH, H // 2), jnp.int32),
            pltpu.VMEM((GCH, H // 2), jnp.int32),
            pltpu.SemaphoreType.DMA,
            pltpu.SemaphoreType.DMA,
        ],
    )
    def _gather(p_hbm, src_hbm, g_hbm, idx0, idx1, buf0, buf1, gs0, gs1):
        c = lax.axis_index("c")
        s = lax.axis_index("s")
        wid = c * NS + s
        start = wid * per_w + jnp.minimum(wid, rem)
        nm1 = per_w - 1 + (wid < rem).astype(jnp.int32)
        idxb = (idx0, idx1)
        bufb = (buf0, buf1)
        gsem = (gs0, gs1)

        def gbase(j):
            return (start + jnp.minimum(j, nm1)) * GCH

        def load_idx(j, b):
            pltpu.sync_copy(src_hbm.at[pl.ds(gbase(j), GCH)], idxb[b])

        def fire(b):
            pltpu.async_copy(p_hbm.at[idxb[b]], bufb[b], gsem[b])

        def drain_store(j, b):
            pltpu.make_async_copy(p_hbm.at[idxb[b]], bufb[b], gsem[b]).wait()
            pltpu.sync_copy(bufb[b], g_hbm.at[pl.ds(gbase(j), GCH)])

        load_idx(0, 0)
        fire(0)
        load_idx(1, 1)

        def body(jj, carry):
            fire(1)
            drain_store(2 * jj, 0)
            load_idx(2 * jj + 2, 0)
            fire(0)
            drain_store(2 * jj + 1, 1)
            load_idx(2 * jj + 3, 1)
            return carry

        lax.fori_loop(0, (steps - 1) // 2, body, 0)
        drain_store(steps - 1, 0)

    return _gather


EH = E // 2
_gather_half = _make_gather(EH, 19, 17, 21)


@functools.partial(
    pl.kernel,
    out_type=jax.ShapeDtypeStruct((N, H), jnp.float32),
    mesh=_sc_mesh,
    scratch_types=[
        pltpu.VMEM((SCH,), jnp.int32),
        pltpu.VMEM((SCH,), jnp.int32),
        pltpu.VMEM((SCH, CG), jnp.float32),
        pltpu.VMEM((SCH, CG), jnp.float32),
        pltpu.VMEM_SHARED((N, CG), jnp.float32),
        pltpu.SemaphoreType.DMA,
        pltpu.SemaphoreType.DMA,
        pltpu.SemaphoreType.DMA,
        pltpu.SemaphoreType.DMA,
    ],
)
def _scatter(h3a_hbm, h3b_hbm, dst_hbm, zeros_hbm, out_hbm,
             idx0, idx1, dat0, dat1, acc_sh, is0, is1, ds0, ds1):
    c = lax.axis_index("c")
    s = lax.axis_index("s")
    last = NS * NPT
    idxb = (idx0, idx1)
    datb = (dat0, dat1)
    isem = (is0, is1)
    dsem = (ds0, ds1)

    for gi in range(2):
        col0 = (c * 2 + gi) * CG

        def run_tile(h3_hbm, lbase, gbase):
            def issue(j, b):
                o = j * SCH
                pltpu.async_copy(dst_hbm.at[pl.ds(gbase + o, SCH)],
                                 idxb[b], isem[b])
                pltpu.async_copy(h3_hbm.at[pl.ds(lbase + o, SCH), pl.ds(col0, CG)],
                                 datb[b], dsem[b])

            def drain_scatter(b):
                pltpu.make_async_copy(dst_hbm.at[pl.ds(0, SCH)], idxb[b], isem[b]).wait()
                pltpu.make_async_copy(h3_hbm.at[pl.ds(0, SCH), pl.ds(0, CG)],
                                      datb[b], dsem[b]).wait()
                pltpu.sync_copy(datb[b], acc_sh.at[idxb[b]], add=True)

            issue(0, 0)

            def body(jj, carry):
                issue(2 * jj + 1, 1)
                drain_scatter(0)
                issue(2 * jj + 2, 0)
                drain_scatter(1)
                return carry

            lax.fori_loop(0, (SPT - 1) // 2, body, 0)
            drain_scatter(0)

        pltpu.sync_copy(zeros_hbm.at[pl.ds(s * NPT, NPT)],
                        acc_sh.at[pl.ds(s * NPT, NPT)])

        @pl.when(s == NS - 1)
        def _():
            pltpu.sync_copy(zeros_hbm.at[pl.ds(last, N - last)],
                            acc_sh.at[pl.ds(last, N - last)])

        plsc.subcore_barrier()

        @pl.when(s < NS // 2)
        def _():
            run_tile(h3a_hbm, s * SPT * SCH, s * SPT * SCH)

        @pl.when(s >= NS // 2)
        def _():
            run_tile(h3b_hbm, (s - NS // 2) * SPT * SCH, s * SPT * SCH)

        plsc.subcore_barrier()
        pltpu.sync_copy(acc_sh.at[pl.ds(s * NPT, NPT)],
                        out_hbm.at[pl.ds(s * NPT, NPT), pl.ds(col0, CG)])

        @pl.when(s == NS - 1)
        def _():
            pltpu.sync_copy(acc_sh.at[pl.ds(last, N - last)],
                            out_hbm.at[pl.ds(last, N - last), pl.ds(col0, CG)])

        plsc.subcore_barrier()


def kernel(x, edge_index, edge_attr, embed,
           W1, b1, g1, be1, W2, b2, g2, be2, W3, b3, g3, be3):
    src = edge_index[0]
    dst = edge_index[1]

    w1aT = jnp.pad(W1[:, :D_IN].T, ((0, 1), (0, 0)))
    w1bT = W1[:, D_IN:D_IN + 32].T
    w1cT = W1[:, D_IN + 32:].T
    b1b = jnp.broadcast_to(b1, (8, H))
    w2Tp = (g1[:, None] * W2.T).astype(jnp.bfloat16)
    w3Tp = (g2[:, None] * W3.T).astype(jnp.bfloat16)
    c2 = jnp.sum(w2Tp.astype(jnp.float32), axis=0)
    c3 = jnp.sum(w3Tp.astype(jnp.float32), axis=0)
    b2p = be1 @ W2.T + b2
    b3p = be2 @ W3.T + b3
    vecs = jnp.stack([b2p, c2, b3p, c3, g3, be3, b2, b3])

    p = pl.pallas_call(
        _node_body,
        grid=(N // BN,),
        in_specs=[
            pl.BlockSpec((BN, 256), lambda i: (i, 0)),
            pl.BlockSpec((256, H), lambda i: (0, 0)),
            pl.BlockSpec((8, 32), lambda i: (0, 0)),
            pl.BlockSpec((32, H), lambda i: (0, 0)),
            pl.BlockSpec((8, H), lambda i: (0, 0)),
        ],
        out_specs=pl.BlockSpec((BN, H), lambda i: (i, 0)),
        out_shape=jax.ShapeDtypeStruct((N, H), jnp.bfloat16),
    )(x, w1aT, embed, w1bT, b1b)
    p = lax.bitcast_convert_type(
        jnp.stack([p[:, :H // 2], p[:, H // 2:]], axis=-1), jnp.int32)

    def mlp(g2d, attr):
        return pl.pallas_call(
            _mlp_body,
            grid=(EH // BE,),
            in_specs=[
                pl.BlockSpec((BE, H // 2), lambda i: (i, 0)),
                pl.BlockSpec((BE, 16), lambda i: (i, 0)),
                pl.BlockSpec((16, H), lambda i: (0, 0)),
                pl.BlockSpec((H, H), lambda i: (0, 0)),
                pl.BlockSpec((H, H), lambda i: (0, 0)),
                pl.BlockSpec((8, H), lambda i: (0, 0)),
            ],
            out_specs=pl.BlockSpec((BE, H), lambda i: (i, 0)),
            out_shape=jax.ShapeDtypeStruct((EH, H), jnp.float32),
        )(g2d, attr, w1cT, w2Tp, w3Tp, vecs)

    ga = _gather_half(p, src[:EH])
    gb = _gather_half(p, src[EH:])
    h3a = mlp(ga, edge_attr[:EH])
    h3b = mlp(gb, edge_attr[EH:])

    zeros = jnp.zeros((N, CG), jnp.float32)
    out = _scatter(h3a, h3b, dst, zeros)
    return out


# speedup vs baseline: 1.0200x; 1.0200x over previous
import functools

import jax
import jax.numpy as jnp
from jax import lax
from jax.experimental import pallas as pl
from jax.experimental.pallas import tpu as pltpu
from jax.experimental.pallas import tpu_sc as plsc

N = 10000
E = 160000
H = 512
D_IN = 255
NC, NS = 2, 16
NW = NC * NS
GCH = 80
SCH = 80
SPT = 125
CG = 128
NG = H // CG
NPT = 624
BN = 1000
BE = 1600


def _ln(h, g, b):
    mu = jnp.mean(h, axis=-1, keepdims=True)
    var = jnp.mean((h - mu) ** 2, axis=-1, keepdims=True)
    return (h - mu) * lax.rsqrt(var + 1e-5) * g + b


def _node_body(x_ref, w1a_ref, embed_ref, w1bT_ref, b1_ref, p_ref):
    xb = x_ref[...]
    q = jnp.dot(embed_ref[...], w1bT_ref[...],
                preferred_element_type=jnp.float32)
    et = xb[:, 255].astype(jnp.int32).reshape(BN, 1)
    onehot = (et == lax.broadcasted_iota(jnp.int32, (1, 8), 1)).astype(jnp.float32)
    p = jnp.dot(xb, w1a_ref[...], preferred_element_type=jnp.float32)
    p = p + jnp.dot(onehot, q, preferred_element_type=jnp.float32)
    p_ref[...] = (p + b1_ref[0:1, :]).astype(jnp.bfloat16)


def _stats(r):
    mu = jnp.mean(r, axis=-1, keepdims=True)
    m2 = jnp.mean(r * r, axis=-1, keepdims=True)
    inv = lax.rsqrt(m2 - mu * mu + 1e-5)
    return mu, inv


def _mlp_body(g_ref, attr_ref, w1cT_ref, w2T_ref, w3T_ref, vecs_ref, h3_ref):
    gw = g_ref[...]
    glo = pltpu.unpack_elementwise(gw, index=0, packed_dtype=jnp.bfloat16,
                                   unpacked_dtype=jnp.float32)
    ghi = pltpu.unpack_elementwise(gw, index=1, packed_dtype=jnp.bfloat16,
                                   unpacked_dtype=jnp.float32)
    gfull = jnp.concatenate([glo, ghi], axis=-1)
    pre1 = gfull + jnp.dot(attr_ref[...], w1cT_ref[...],
                           preferred_element_type=jnp.float32)
    r = jax.nn.relu(pre1)
    mu, inv = _stats(r)
    t = jnp.dot(r.astype(jnp.bfloat16), w2T_ref[...],
                preferred_element_type=jnp.float32)
    pre2 = t * inv - (mu * inv) * vecs_ref[1:2, :] + vecs_ref[0:1, :]
    r = jax.nn.relu(pre2)
    mu, inv = _stats(r)
    t = jnp.dot(r.astype(jnp.bfloat16), w3T_ref[...],
                preferred_element_type=jnp.float32)
    pre3 = t * inv - (mu * inv) * vecs_ref[3:4, :] + vecs_ref[2:3, :]
    r = jax.nn.relu(pre3)
    mu, inv = _stats(r)
    h3_ref[...] = (r - mu) * (inv * vecs_ref[4:5, :]) + vecs_ref[5:6, :]


_sc_mesh = plsc.VectorSubcoreMesh(core_axis_name="c", subcore_axis_name="s")


def _make_gather(n_edges, per_w, rem, steps):
    @functools.partial(
        pl.kernel,
        out_type=jax.ShapeDtypeStruct((n_edges, H // 2), jnp.int32),
        mesh=_sc_mesh,
        scratch_types=[
            pltpu.VMEM((GCH,), jnp.int32),
            pltpu.VMEM((GCH,), jnp.int32),
            pltpu.VMEM((GCH, H // 2), jnp.int32),
            pltpu.VMEM((GCH, H // 2), jnp.int32),
            pltpu.SemaphoreType.DMA,
            pltpu.SemaphoreType.DMA,
        ],
    )
    def _gather(p_hbm, src_hbm, g_hbm, idx0, idx1, buf0, buf1, gs0, gs1):
        c = lax.axis_index("c")
        s = lax.axis_index("s")
        wid = c * NS + s
        start = wid * per_w + jnp.minimum(wid, rem)
        nm1 = per_w - 1 + (wid < rem).astype(jnp.int32)
        idxb = (idx0, idx1)
        bufb = (buf0, buf1)
        gsem = (gs0, gs1)

        def gbase(j):
            return (start + jnp.minimum(j, nm1)) * GCH

        def load_idx(j, b):
            pltpu.sync_copy(src_hbm.at[pl.ds(gbase(j), GCH)], idxb[b])

        def fire(b):
            pltpu.async_copy(p_hbm.at[idxb[b]], bufb[b], gsem[b])

        def drain_store(j, b):
            pltpu.make_async_copy(p_hbm.at[idxb[b]], bufb[b], gsem[b]).wait()
            pltpu.sync_copy(bufb[b], g_hbm.at[pl.ds(gbase(j), GCH)])

        load_idx(0, 0)
        fire(0)
        load_idx(1, 1)

        def body(jj, carry):
            fire(1)
            drain_store(2 * jj, 0)
            load_idx(2 * jj + 2, 0)
            fire(0)
            drain_store(2 * jj + 1, 1)
            load_idx(2 * jj + 3, 1)
            return carry

        lax.fori_loop(0, (steps - 1) // 2, body, 0)
        drain_store(steps - 1, 0)

    return _gather


EH = E // 2
_gather_half = _make_gather(EH, 31, 8, 33)


@functools.partial(
    pl.kernel,
    out_type=jax.ShapeDtypeStruct((N, H), jnp.float32),
    mesh=_sc_mesh,
    scratch_types=[
        pltpu.VMEM((SCH,), jnp.int32),
        pltpu.VMEM((SCH,), jnp.int32),
        pltpu.VMEM((SCH, CG), jnp.float32),
        pltpu.VMEM((SCH, CG), jnp.float32),
        pltpu.VMEM_SHARED((N, CG), jnp.float32),
        pltpu.SemaphoreType.DMA,
        pltpu.SemaphoreType.DMA,
        pltpu.SemaphoreType.DMA,
        pltpu.SemaphoreType.DMA,
    ],
)
def _scatter(h3a_hbm, h3b_hbm, dst_hbm, zeros_hbm, out_hbm,
             idx0, idx1, dat0, dat1, acc_sh, is0, is1, ds0, ds1):
    c = lax.axis_index("c")
    s = lax.axis_index("s")
    last = NS * NPT
    idxb = (idx0, idx1)
    datb = (dat0, dat1)
    isem = (is0, is1)
    dsem = (ds0, ds1)

    for gi in range(2):
        col0 = (c * 2 + gi) * CG

        def run_tile(h3_hbm, lbase, gbase):
            def issue(j, b):
                o = j * SCH
                pltpu.async_copy(dst_hbm.at[pl.ds(gbase + o, SCH)],
                                 idxb[b], isem[b])
                pltpu.async_copy(h3_hbm.at[pl.ds(lbase + o, SCH), pl.ds(col0, CG)],
                                 datb[b], dsem[b])

            def drain_scatter(b):
                pltpu.make_async_copy(dst_hbm.at[pl.ds(0, SCH)], idxb[b], isem[b]).wait()
                pltpu.make_async_copy(h3_hbm.at[pl.ds(0, SCH), pl.ds(0, CG)],
                                      datb[b], dsem[b]).wait()
                pltpu.sync_copy(datb[b], acc_sh.at[idxb[b]], add=True)

            issue(0, 0)

            def body(jj, carry):
                issue(2 * jj + 1, 1)
                drain_scatter(0)
                issue(2 * jj + 2, 0)
                drain_scatter(1)
                return carry

            lax.fori_loop(0, (SPT - 1) // 2, body, 0)
            drain_scatter(0)

        pltpu.sync_copy(zeros_hbm.at[pl.ds(s * NPT, NPT)],
                        acc_sh.at[pl.ds(s * NPT, NPT)])

        @pl.when(s == NS - 1)
        def _():
            pltpu.sync_copy(zeros_hbm.at[pl.ds(last, N - last)],
                            acc_sh.at[pl.ds(last, N - last)])

        plsc.subcore_barrier()

        @pl.when(s < NS // 2)
        def _():
            run_tile(h3a_hbm, s * SPT * SCH, s * SPT * SCH)

        @pl.when(s >= NS // 2)
        def _():
            run_tile(h3b_hbm, (s - NS // 2) * SPT * SCH, s * SPT * SCH)

        plsc.subcore_barrier()
        pltpu.sync_copy(acc_sh.at[pl.ds(s * NPT, NPT)],
                        out_hbm.at[pl.ds(s * NPT, NPT), pl.ds(col0, CG)])

        @pl.when(s == NS - 1)
        def _():
            pltpu.sync_copy(acc_sh.at[pl.ds(last, N - last)],
                            out_hbm.at[pl.ds(last, N - last), pl.ds(col0, CG)])

        plsc.subcore_barrier()


def kernel(x, edge_index, edge_attr, embed,
           W1, b1, g1, be1, W2, b2, g2, be2, W3, b3, g3, be3):
    src = edge_index[0]
    dst = edge_index[1]

    w1aT = jnp.pad(W1[:, :D_IN].T, ((0, 1), (0, 0)))
    w1bT = W1[:, D_IN:D_IN + 32].T
    w1cT = W1[:, D_IN + 32:].T
    b1b = jnp.broadcast_to(b1, (8, H))
    w2Tp = (g1[:, None] * W2.T).astype(jnp.bfloat16)
    w3Tp = (g2[:, None] * W3.T).astype(jnp.bfloat16)
    c2 = jnp.sum(w2Tp.astype(jnp.float32), axis=0)
    c3 = jnp.sum(w3Tp.astype(jnp.float32), axis=0)
    b2p = be1 @ W2.T + b2
    b3p = be2 @ W3.T + b3
    vecs = jnp.stack([b2p, c2, b3p, c3, g3, be3, b2, b3])

    p = pl.pallas_call(
        _node_body,
        grid=(N // BN,),
        in_specs=[
            pl.BlockSpec((BN, 256), lambda i: (i, 0)),
            pl.BlockSpec((256, H), lambda i: (0, 0)),
            pl.BlockSpec((8, 32), lambda i: (0, 0)),
            pl.BlockSpec((32, H), lambda i: (0, 0)),
            pl.BlockSpec((8, H), lambda i: (0, 0)),
        ],
        out_specs=pl.BlockSpec((BN, H), lambda i: (i, 0)),
        out_shape=jax.ShapeDtypeStruct((N, H), jnp.bfloat16),
    )(x, w1aT, embed, w1bT, b1b)
    p = lax.bitcast_convert_type(
        jnp.stack([p[:, :H // 2], p[:, H // 2:]], axis=-1), jnp.int32)

    def mlp(g2d, attr):
        return pl.pallas_call(
            _mlp_body,
            grid=(EH // BE,),
            in_specs=[
                pl.BlockSpec((BE, H // 2), lambda i: (i, 0)),
                pl.BlockSpec((BE, 16), lambda i: (i, 0)),
                pl.BlockSpec((16, H), lambda i: (0, 0)),
                pl.BlockSpec((H, H), lambda i: (0, 0)),
                pl.BlockSpec((H, H), lambda i: (0, 0)),
                pl.BlockSpec((8, H), lambda i: (0, 0)),
            ],
            out_specs=pl.BlockSpec((BE, H), lambda i: (i, 0)),
            out_shape=jax.ShapeDtypeStruct((EH, H), jnp.float32),
        )(g2d, attr, w1cT, w2Tp, w3Tp, vecs)

    ga = _gather_half(p, src[:EH])
    gb = _gather_half(p, src[EH:])
    h3a = mlp(ga, edge_attr[:EH])
    h3b = mlp(gb, edge_attr[EH:])

    zeros = jnp.zeros((N, CG), jnp.float32)
    out = _scatter(h3a, h3b, dst, zeros)
    return out
